# Initial kernel scaffold; baseline (speedup 1.0000x reference)
import jax
import jax.numpy as jnp
from jax.experimental import pallas as pl


def kernel(x, codebook, scale):
    raise NotImplementedError("write your pallas kernel here")


# speedup vs baseline: 1.6421x; 1.6421x over previous
import jax
import jax.numpy as jnp
from jax.experimental import pallas as pl
from jax.experimental.pallas import tpu as pltpu


_M = 4


def _drq_kernel(scale_ref, x_ref, cb_ref, codes_ref, loss_ref):
    i = pl.program_id(0)
    x = x_ref[...]
    cb = cb_ref[...]
    n_total = pl.num_programs(0) * x.shape[0]
    inv_nd = 1.0 / (n_total * x.shape[1])

    residual = x
    qsoft = jnp.zeros_like(x)
    qhard = jnp.zeros_like(x)
    part = jnp.float32(0.0)
    for m in range(_M):
        s = scale_ref[m]
        cbm = cb * s
        cn = jnp.sum(cbm * cbm, axis=1)
        rn = jnp.sum(residual * residual, axis=1, keepdims=True)
        g = jax.lax.dot_general(
            residual, cbm, (((1,), (1,)), ((), ())),
            preferred_element_type=jnp.float32)
        dist = -(rn - 2.0 * g + cn[None, :])
        mx = jnp.max(dist, axis=1, keepdims=True)
        e = jnp.exp(dist - mx)
        w = e / jnp.sum(e, axis=1, keepdims=True)
        soft = jnp.dot(w, cbm, preferred_element_type=jnp.float32)
        code = jnp.argmax(dist, axis=1)
        codes_ref[:, m] = code
        oh = (jax.lax.broadcasted_iota(jnp.int32, dist.shape, 1)
              == code[:, None]).astype(jnp.float32)
        hard = jnp.dot(oh, cbm, preferred_element_type=jnp.float32)
        residual = residual - hard
        qsoft = qsoft + soft
        qhard = qhard + hard
        part += 0.1 * jnp.sum((x - qsoft) ** 2) + jnp.sum((x - qhard) ** 2)
    part += 0.1 * jnp.sum((qsoft - qhard) ** 2)

    @pl.when(i == 0)
    def _():
        loss_ref[0] = 0.0

    loss_ref[0] += part * inv_nd


def kernel(x, codebook, scale):
    n, d = x.shape
    k = codebook.shape[0]
    bn = 512
    grid = (n // bn,)
    codes, loss = pl.pallas_call(
        _drq_kernel,
        grid=grid,
        in_specs=[
            pl.BlockSpec(memory_space=pltpu.SMEM),
            pl.BlockSpec((bn, d), lambda i: (i, 0)),
            pl.BlockSpec((k, d), lambda i: (0, 0)),
        ],
        out_specs=[
            pl.BlockSpec((bn, _M), lambda i: (i, 0)),
            pl.BlockSpec(memory_space=pltpu.SMEM),
        ],
        out_shape=[
            jax.ShapeDtypeStruct((n, _M), jnp.int32),
            jax.ShapeDtypeStruct((1,), jnp.float32),
        ],
        compiler_params=pltpu.CompilerParams(
            dimension_semantics=("arbitrary",)),
    )(scale, x, codebook)
    return codes, loss[0]


# speedup vs baseline: 2.7158x; 1.6538x over previous
import functools

import jax
import jax.numpy as jnp
from jax.experimental import pallas as pl
from jax.experimental.pallas import tpu as pltpu


_M = 4
_W = 128


def _drq_kernel(scale_ref, x_ref, cb_ref, codes_ref, loss_ref, tab_ref):
    i = pl.program_id(0)
    k, d = cb_ref.shape

    @pl.when(i == 0)
    def _init():
        cb = cb_ref[...]
        pad = jnp.zeros((k, _W - d - 2), jnp.float32)
        ones = jnp.ones((k, 1), jnp.float32)
        for m in range(_M):
            cbm = cb * scale_ref[m]
            cn = jnp.sum(cbm * cbm, axis=1, keepdims=True)
            tab_ref[m] = jnp.concatenate([cbm, cn, ones, pad], axis=1)
        loss_ref[0] = 0.0

    x = x_ref[...]
    bn = x.shape[0]
    n_total = pl.num_programs(0) * bn
    inv_nd = 1.0 / (n_total * d)

    dotf = functools.partial(
        jax.lax.dot_general, preferred_element_type=jnp.float32)

    neg1 = jnp.full((bn, 1), -1.0, jnp.float32)
    rpad = jnp.zeros((bn, _W - d - 1), jnp.float32)

    residual = x
    qsoft = jnp.zeros_like(x)
    qhard = jnp.zeros_like(x)
    part = jnp.float32(0.0)
    for m in range(_M):
        tab = tab_ref[m]
        r_aug = jnp.concatenate([residual + residual, neg1, rpad], axis=1)
        logits = dotf(r_aug, tab, (((1,), (1,)), ((), ())))
        mx = jnp.max(logits, axis=1, keepdims=True)
        e = jnp.exp(logits - mx)
        se = dotf(e, tab, (((1,), (0,)), ((), ())))
        soft = se[:, :d] / se[:, d + 1:d + 2]
        idxs = jax.lax.broadcasted_iota(jnp.int32, logits.shape, 1)
        masked = jnp.where(logits >= mx, idxs, jnp.int32(k))
        code = jnp.min(masked, axis=1)
        codes_ref[:, m] = code
        oh = (idxs == code[:, None]).astype(jnp.float32)
        hard = dotf(oh, tab, (((1,), (0,)), ((), ())))[:, :d]
        residual = residual - hard
        qsoft = qsoft + soft
        qhard = qhard + hard
        part += 0.1 * jnp.sum((x - qsoft) ** 2) + jnp.sum((x - qhard) ** 2)
    part += 0.1 * jnp.sum((qsoft - qhard) ** 2)

    loss_ref[0] += part * inv_nd


def kernel(x, codebook, scale):
    n, d = x.shape
    k = codebook.shape[0]
    bn = 512
    grid = (n // bn,)
    codes, loss = pl.pallas_call(
        _drq_kernel,
        grid=grid,
        in_specs=[
            pl.BlockSpec(memory_space=pltpu.SMEM),
            pl.BlockSpec((bn, d), lambda i: (i, 0)),
            pl.BlockSpec((k, d), lambda i: (0, 0)),
        ],
        out_specs=[
            pl.BlockSpec((bn, _M), lambda i: (i, 0)),
            pl.BlockSpec(memory_space=pltpu.SMEM),
        ],
        out_shape=[
            jax.ShapeDtypeStruct((n, _M), jnp.int32),
            jax.ShapeDtypeStruct((1,), jnp.float32),
        ],
        scratch_shapes=[pltpu.VMEM((_M, k, _W), jnp.float32)],
        compiler_params=pltpu.CompilerParams(
            dimension_semantics=("arbitrary",)),
    )(scale, x, codebook)
    return codes, loss[0]
